# baseline (speedup 1.0000x reference)
import functools

import jax
import jax.numpy as jnp
from jax import lax
from jax.experimental import pallas as pl
from jax.experimental.pallas import tpu as pltpu
from jax.experimental.pallas import tpu_sc as plsc

NS, L = 16, 16
ROWS, COLS = 64, 8192
ROWS_PER_W = ROWS // NS
CHUNKS = COLS // L

_mesh = plsc.VectorSubcoreMesh(
    core_axis_name="c", subcore_axis_name="s", num_cores=1)


@functools.partial(
    pl.kernel,
    mesh=_mesh,
    out_type=jax.ShapeDtypeStruct((ROWS,), jnp.int32),
    scratch_types=[
        pltpu.VMEM((COLS,), jnp.float32),
        pltpu.VMEM((COLS,), jnp.float32),
        pltpu.VMEM((COLS,), jnp.float32),
        pltpu.VMEM((COLS,), jnp.float32),
        pltpu.VMEM((L,), jnp.int32),
        pltpu.VMEM((NS * L,), jnp.int32),
        pltpu.VMEM((ROWS,), jnp.int32),
        pltpu.VMEM_SHARED((NS * L,), jnp.int32),
        pltpu.SemaphoreType.DMA,
        pltpu.SemaphoreType.DMA,
        pltpu.SemaphoreType.DMA,
        pltpu.SemaphoreType.DMA,
        pltpu.SemaphoreType.DMA,
        pltpu.SemaphoreType.DMA,
        pltpu.SemaphoreType.DMA,
    ],
)
def _argmax_sc(in_hbm, out_hbm, buf0, buf1, buf2, buf3, stage, gath, cbuf,
               shared, sem0a, sem0b, sem0c, sem0d, sem1, sem2, sem3):
    s = lax.axis_index("s")
    r0 = s * ROWS_PER_W
    q = COLS // 4
    cp0 = [
        pltpu.async_copy(
            in_hbm.at[r0, pl.ds(i * q, q)], buf0.at[pl.ds(i * q, q)], sem)
        for i, sem in enumerate((sem0a, sem0b, sem0c, sem0d))
    ]
    cp1 = pltpu.async_copy(in_hbm.at[r0 + 1], buf1, sem1)
    lanes = lax.iota(jnp.int32, 16)
    big = jnp.int32(2**31 - 1)

    def shuffle(x, k):
        return x.at[lanes ^ k].get(mode="promise_in_bounds")

    neg_inf = jnp.full((L,), -jnp.inf, jnp.float32)

    def argmax_row(buf, copies=None):
        unroll = 16
        steps = CHUNKS // unroll

        def body(t, carry):
            maxvs, maxts = (list(x) for x in carry)
            tvec = jnp.full((L,), 0, jnp.int32) + t
            base = t * unroll
            for u in range(unroll):
                chunk = buf[pl.ds((base + u) * L, L)]
                old = maxvs[u]
                pred = chunk > old
                maxvs[u] = jnp.maximum(old, chunk)
                maxts[u] = jnp.where(pred, tvec, maxts[u])
            return tuple(maxvs), tuple(maxts)

        carry = ((neg_inf,) * unroll,
                 (jnp.full((L,), 0, jnp.int32),) * unroll)
        if copies is None:
            carry = lax.fori_loop(0, steps, body, carry)
        else:
            per = steps // len(copies)
            for i, cp in enumerate(copies):
                cp.wait()
                carry = lax.fori_loop(i * per, (i + 1) * per, body, carry)
        maxvs, maxts = carry
        m = maxvs[0]
        for u in range(1, unroll):
            m = jnp.maximum(m, maxvs[u])
        for k in (1, 2, 4, 8):
            m = jnp.maximum(m, shuffle(m, k))
        cand = None
        for u in range(unroll):
            idx = (maxts[u] << 8) + (lanes + u * L)
            c_u = jnp.where(maxvs[u] == m, idx, big)
            cand = c_u if cand is None else jnp.minimum(cand, c_u)
        for k in (1, 2, 4, 8):
            cand = jnp.minimum(cand, shuffle(cand, k))
        return cand

    a0 = argmax_row(buf0, cp0)
    cp2 = pltpu.async_copy(in_hbm.at[r0 + 2], buf2, sem2)
    cp1.wait()
    a1 = argmax_row(buf1)
    cp3 = pltpu.async_copy(in_hbm.at[r0 + 3], buf3, sem3)
    cp2.wait()
    a2 = argmax_row(buf2)
    cp3.wait()
    a3 = argmax_row(buf3)
    res = jnp.zeros((L,), jnp.int32)
    for i, a in enumerate((a0, a1, a2, a3)):
        res = jnp.where(lanes == (ROWS_PER_W * s + i) % L, a, res)
    stage[...] = res
    pltpu.sync_copy(stage, shared.at[pl.ds(s * L, L)])
    plsc.subcore_barrier()

    @pl.when(s == 0)
    def _():
        pltpu.sync_copy(shared, gath)
        for g in range(4):
            acc = gath[pl.ds(4 * g * L, L)]
            for t in range(1, 4):
                acc = acc + gath[pl.ds((4 * g + t) * L, L)]
            cbuf[pl.ds(g * L, L)] = acc
        pltpu.sync_copy(cbuf, out_hbm)


def kernel(inputs):
    return _argmax_sc(inputs).astype(jnp.int64)


# speedup vs baseline: 1.0129x; 1.0129x over previous
import functools

import jax
import jax.numpy as jnp
from jax import lax
from jax.experimental import pallas as pl
from jax.experimental.pallas import tpu as pltpu
from jax.experimental.pallas import tpu_sc as plsc

NC, NS, L = 2, 16, 16
NW = NC * NS
ROWS, COLS = 64, 8192
ROWS_PER_W = ROWS // NW
CHUNKS = COLS // L
ROWS_PER_C = ROWS // NC

_mesh = plsc.VectorSubcoreMesh(core_axis_name="c", subcore_axis_name="s")


@functools.partial(
    pl.kernel,
    mesh=_mesh,
    out_type=jax.ShapeDtypeStruct((ROWS,), jnp.int32),
    scratch_types=[
        pltpu.VMEM((COLS,), jnp.float32),
        pltpu.VMEM((COLS,), jnp.float32),
        pltpu.VMEM((L,), jnp.int32),
        pltpu.VMEM((NS * L,), jnp.int32),
        pltpu.VMEM((ROWS_PER_C,), jnp.int32),
        pltpu.VMEM_SHARED((NS * L,), jnp.int32),
        pltpu.SemaphoreType.DMA,
        pltpu.SemaphoreType.DMA,
        pltpu.SemaphoreType.DMA,
        pltpu.SemaphoreType.DMA,
        pltpu.SemaphoreType.DMA,
    ],
)
def _argmax_sc(in_hbm, out_hbm, buf0, buf1, stage, gath, cbuf, shared,
               sem0a, sem0b, sem0c, sem0d, sem1):
    c = lax.axis_index("c")
    s = lax.axis_index("s")
    r0 = (c * NS + s) * ROWS_PER_W
    q = COLS // 4
    cp0 = [
        pltpu.async_copy(
            in_hbm.at[r0, pl.ds(i * q, q)], buf0.at[pl.ds(i * q, q)], sem)
        for i, sem in enumerate((sem0a, sem0b, sem0c, sem0d))
    ]
    cp1 = pltpu.async_copy(in_hbm.at[r0 + 1], buf1, sem1)
    lanes = lax.iota(jnp.int32, 16)
    big = jnp.int32(2**31 - 1)

    def shuffle(x, k):
        return x.at[lanes ^ k].get(mode="promise_in_bounds")

    neg_inf = jnp.full((L,), -jnp.inf, jnp.float32)

    def argmax_row(buf, copies=None):
        unroll = 16
        steps = CHUNKS // unroll

        def body(t, carry):
            maxvs, maxts = (list(x) for x in carry)
            tvec = jnp.full((L,), 0, jnp.int32) + t
            base = t * unroll
            for u in range(unroll):
                chunk = buf[pl.ds((base + u) * L, L)]
                old = maxvs[u]
                pred = chunk > old
                maxvs[u] = jnp.maximum(old, chunk)
                maxts[u] = jnp.where(pred, tvec, maxts[u])
            return tuple(maxvs), tuple(maxts)

        carry = ((neg_inf,) * unroll, (jnp.full((L,), 0, jnp.int32),) * unroll)
        if copies is None:
            carry = plsc.parallel_loop(0, steps, 1, carry=carry)(body)
        else:
            per = steps // len(copies)
            for i, cp in enumerate(copies):
                cp.wait()
                carry = plsc.parallel_loop(
                    i * per, (i + 1) * per, 1, carry=carry)(body)
        maxvs, maxts = carry
        m = maxvs[0]
        for u in range(1, unroll):
            m = jnp.maximum(m, maxvs[u])
        for k in (1, 2, 4, 8):
            m = jnp.maximum(m, shuffle(m, k))
        cand = None
        for u in range(unroll):
            idx = (maxts[u] << 8) + (lanes + u * L)
            c_u = jnp.where(maxvs[u] == m, idx, big)
            cand = c_u if cand is None else jnp.minimum(cand, c_u)
        for k in (1, 2, 4, 8):
            cand = jnp.minimum(cand, shuffle(cand, k))
        return cand

    a0 = argmax_row(buf0, cp0)
    cp1.wait()
    a1 = argmax_row(buf1)
    p0 = (2 * s) % L
    stage[...] = jnp.where(lanes == p0, a0, jnp.where(lanes == p0 + 1, a1, 0))
    pltpu.sync_copy(stage, shared.at[pl.ds(s * L, L)])
    plsc.subcore_barrier()

    @pl.when(s == 0)
    def _():
        pltpu.sync_copy(shared, gath)
        for h in range(2):
            acc = gath[pl.ds(8 * h * L, L)]
            for t in range(1, 8):
                acc = acc + gath[pl.ds((8 * h + t) * L, L)]
            cbuf[pl.ds(h * L, L)] = acc
        pltpu.sync_copy(cbuf, out_hbm.at[pl.ds(c * ROWS_PER_C, ROWS_PER_C)])


def kernel(inputs):
    return _argmax_sc(inputs).astype(jnp.int64)
